# Initial kernel scaffold; baseline (speedup 1.0000x reference)
import jax
import jax.numpy as jnp
from jax.experimental import pallas as pl


def kernel(seq, edge_index, edge_weight, W):
    raise NotImplementedError("write your pallas kernel here")


# speedup vs baseline: 2.9105x; 2.9105x over previous
import functools

import jax
import jax.numpy as jnp
from jax import lax
from jax.experimental import pallas as pl
from jax.experimental.pallas import tpu as pltpu
from jax.experimental.pallas import tpu_sc as plsc

N_NODES = 10000
FT = 128

NC = 2
NS = 16
NW = NC * NS
CHUNK = 128
N_CHUNKS = 80
E_PER_W = CHUNK * N_CHUNKS
EPAD = NW * E_PER_W
NPAD = 10240
ROWS_PER_TILE = NPAD // NS


def _spmm_body(seq_hbm, src_hbm, dst_hbm, w_hbm, zeros_hbm, out_hbm,
               acc_shared, src_v, dst_v, w_v, rows_v, gsem):
  c = lax.axis_index("c")
  s = lax.axis_index("s")
  wid = s * NC + c

  pltpu.sync_copy(zeros_hbm.at[pl.ds(s * ROWS_PER_TILE, ROWS_PER_TILE)],
                  acc_shared.at[pl.ds(s * ROWS_PER_TILE, ROWS_PER_TILE)])
  pltpu.sync_copy(src_hbm.at[wid], src_v)
  pltpu.sync_copy(dst_hbm.at[wid], dst_v)
  pltpu.sync_copy(w_hbm.at[wid], w_v)
  plsc.subcore_barrier()

  def chunk_body(i, carry):
    pltpu.async_copy(seq_hbm.at[src_v.at[i]], rows_v, gsem).wait()
    def group_body(g, carry2):
      w16 = w_v[pl.ds(i * CHUNK + g * 16, 16)]
      for e in range(16):
        wb = lax.gather(
            w16, jnp.full((16, 1), e, dtype=jnp.int32),
            lax.GatherDimensionNumbers(offset_dims=(),
                                       collapsed_slice_dims=(0,),
                                       start_index_map=(0,)),
            slice_sizes=(1,),
            mode=lax.GatherScatterMode.PROMISE_IN_BOUNDS)
        row = g * 16 + e
        for f in range(FT // 16):
          sl = pl.ds(f * 16, 16)
          rows_v[row, sl] = rows_v[row, sl] * wb
      return carry2

    lax.fori_loop(0, CHUNK // 16, group_body, 0)

    pltpu.sync_copy(rows_v, acc_shared.at[dst_v.at[i]], add=True)
    return carry

  lax.fori_loop(0, N_CHUNKS, chunk_body, 0)
  plsc.subcore_barrier()

  pltpu.sync_copy(acc_shared.at[pl.ds(s * ROWS_PER_TILE, ROWS_PER_TILE)],
                  out_hbm.at[c, pl.ds(s * ROWS_PER_TILE, ROWS_PER_TILE)])


_spmm = pl.kernel(
    _spmm_body,
    out_type=jax.ShapeDtypeStruct((NC, NPAD, FT), jnp.float32),
    mesh=plsc.VectorSubcoreMesh(core_axis_name="c", subcore_axis_name="s"),
    scratch_types=[
        pltpu.VMEM_SHARED((NPAD, FT), jnp.float32),
        pltpu.VMEM((N_CHUNKS, CHUNK), jnp.int32),
        pltpu.VMEM((N_CHUNKS, CHUNK), jnp.int32),
        pltpu.VMEM((E_PER_W,), jnp.float32),
        pltpu.VMEM((CHUNK, FT), jnp.float32),
        pltpu.SemaphoreType.DMA,
    ],
)


def _mm_body(p0_ref, p1_ref, w_ref, o_ref):
  p = p0_ref[...] + p1_ref[...]
  o_ref[...] = jnp.maximum(
      jnp.dot(p, w_ref[...], preferred_element_type=jnp.float32), 0.0)


def _matmul_relu(p0, p1, W):
  blk = 1000
  return pl.pallas_call(
      _mm_body,
      grid=(N_NODES // blk,),
      in_specs=[
          pl.BlockSpec((blk, FT), lambda i: (i, 0)),
          pl.BlockSpec((blk, FT), lambda i: (i, 0)),
          pl.BlockSpec((FT, FT), lambda i: (0, 0)),
      ],
      out_specs=pl.BlockSpec((blk, FT), lambda i: (i, 0)),
      out_shape=jax.ShapeDtypeStruct((N_NODES, FT), jnp.float32),
  )(p0, p1, W)


@jax.jit
def kernel(seq, edge_index, edge_weight, W):
  n_edges = edge_index.shape[1]
  src = edge_index[0].astype(jnp.int32)
  dst = edge_index[1].astype(jnp.int32)
  w = edge_weight.astype(jnp.float32)

  pad = EPAD - n_edges
  src = jnp.pad(src, (0, pad)).reshape(NW, N_CHUNKS, CHUNK)
  dst = jnp.pad(dst, (0, pad)).reshape(NW, N_CHUNKS, CHUNK)
  w = jnp.pad(w, (0, pad)).reshape(NW, E_PER_W)

  zeros = jnp.zeros((NPAD, FT), jnp.float32)
  partial = _spmm(seq, src, dst, w, zeros)
  return _matmul_relu(partial[0], partial[1], W)


# speedup vs baseline: 3.4859x; 1.1977x over previous
import jax
import jax.numpy as jnp
from jax import lax
from jax.experimental import pallas as pl
from jax.experimental.pallas import tpu as pltpu
from jax.experimental.pallas import tpu_sc as plsc

N_NODES = 10000
FT = 128

NC = 2
NS = 16
NW = NC * NS
CHUNK = 64
NBUF = 4
SG_CH = 16
NSG = 10
N_CHUNKS = SG_CH * NSG
E_PER_W = CHUNK * N_CHUNKS
E_PER_SG = CHUNK * SG_CH
EPAD = NW * E_PER_W
NPAD = 10240
ROWS_PER_TILE = NPAD // NS


def _bcast16(vec, lane):
  return lax.gather(
      vec, jnp.full((16, 1), lane, dtype=jnp.int32),
      lax.GatherDimensionNumbers(offset_dims=(),
                                 collapsed_slice_dims=(0,),
                                 start_index_map=(0,)),
      slice_sizes=(1,),
      mode=lax.GatherScatterMode.PROMISE_IN_BOUNDS)


def _spmm_body(seq_hbm, src_hbm, dst_hbm, w_hbm, zeros_hbm, out_hbm,
               acc_shared, src_sg, dst_sg, w_sg, rows_v,
               gsems, ssems, srcsems, dstsems, wsems):
  c = lax.axis_index("c")
  s = lax.axis_index("s")
  wid = s * NC + c

  pltpu.sync_copy(zeros_hbm.at[pl.ds(s * ROWS_PER_TILE, ROWS_PER_TILE)],
                  acc_shared.at[pl.ds(s * ROWS_PER_TILE, ROWS_PER_TILE)])
  pltpu.sync_copy(src_hbm.at[wid, 0], src_sg.at[0])
  pltpu.sync_copy(dst_hbm.at[wid, 0], dst_sg.at[0])
  pltpu.sync_copy(w_hbm.at[wid, 0], w_sg.at[0])
  plsc.subcore_barrier()

  def gather_start(sbuf, row, buf):
    pltpu.async_copy(seq_hbm.at[src_sg.at[sbuf, row]], rows_v.at[buf],
                     gsems.at[buf])

  def scale(cis, p2, buf):
    @plsc.parallel_loop(0, CHUNK // 16)
    def group_body(g):
      w16 = w_sg[p2, pl.ds(cis * CHUNK + g * 16, 16)]
      for e in range(16):
        wb = _bcast16(w16, e)
        row = g * 16 + e
        for f in range(FT // 16):
          sl = pl.ds(f * 16, 16)
          rows_v[buf, row, sl] = rows_v[buf, row, sl] * wb

  def chunk_step(ch, cis, p2, b, gather_ahead):
    bprev = (b + NBUF - 1) % NBUF
    pltpu.make_async_copy(seq_hbm.at[src_sg.at[p2, 0]], rows_v.at[b],
                          gsems.at[b]).wait()
    scale(cis, p2, b)
    pltpu.async_copy(rows_v.at[b], acc_shared.at[dst_sg.at[p2, cis]],
                     ssems.at[b], add=True)

    @pl.when(ch >= 1)
    def _():
      pltpu.make_async_copy(rows_v.at[bprev],
                            acc_shared.at[dst_sg.at[p2, cis]],
                            ssems.at[bprev]).wait()

    gather_ahead(bprev)

  for b in range(NBUF - 1):
    gather_start(0, b, b)

  def sg_pair_body(sg2, carry):
    for p2 in range(2):
      sg = sg2 * 2 + p2

      @pl.when(sg + 1 < NSG)
      def _():
        pltpu.async_copy(src_hbm.at[wid, sg + 1], src_sg.at[1 - p2],
                         srcsems.at[1 - p2])
        pltpu.async_copy(dst_hbm.at[wid, sg + 1], dst_sg.at[1 - p2],
                         dstsems.at[1 - p2])
        pltpu.async_copy(w_hbm.at[wid, sg + 1], w_sg.at[1 - p2],
                         wsems.at[1 - p2])

      @pl.when(sg >= 1)
      def _():
        pltpu.make_async_copy(dst_hbm.at[wid, sg], dst_sg.at[p2],
                              dstsems.at[p2]).wait()
        pltpu.make_async_copy(w_hbm.at[wid, sg], w_sg.at[p2],
                              wsems.at[p2]).wait()

      def quad_body(q, carry2):
        for b in range(NBUF):
          cis = q * NBUF + b
          ch = sg * SG_CH + cis
          chunk_step(ch, cis, p2, b,
                     lambda bp, cis=cis: gather_start(p2, cis + 3, bp))
        return carry2

      lax.fori_loop(0, SG_CH // NBUF - 1, quad_body, 0)

      @pl.when(sg + 1 < NSG)
      def _():
        pltpu.make_async_copy(src_hbm.at[wid, sg + 1], src_sg.at[1 - p2],
                              srcsems.at[1 - p2]).wait()

      for b in range(NBUF):
        cis = SG_CH - NBUF + b
        ch = sg * SG_CH + cis

        if b == 0:
          ga = lambda bp: gather_start(p2, SG_CH - 1, bp)
        else:
          def ga(bp, b=b):
            @pl.when(sg + 1 < NSG)
            def _():
              gather_start(1 - p2, b - 1, bp)

        chunk_step(ch, cis, p2, b, ga)
    return carry

  lax.fori_loop(0, NSG // 2, sg_pair_body, 0)
  pltpu.make_async_copy(rows_v.at[NBUF - 1], acc_shared.at[dst_sg.at[0, 0]],
                        ssems.at[NBUF - 1]).wait()
  plsc.subcore_barrier()

  pltpu.sync_copy(acc_shared.at[pl.ds(s * ROWS_PER_TILE, ROWS_PER_TILE)],
                  out_hbm.at[c, pl.ds(s * ROWS_PER_TILE, ROWS_PER_TILE)])


_spmm = pl.kernel(
    _spmm_body,
    out_type=jax.ShapeDtypeStruct((NC, NPAD, FT), jnp.float32),
    mesh=plsc.VectorSubcoreMesh(core_axis_name="c", subcore_axis_name="s"),
    scratch_types=[
        pltpu.VMEM_SHARED((NPAD, FT), jnp.float32),
        pltpu.VMEM((2, SG_CH, CHUNK), jnp.int32),
        pltpu.VMEM((2, SG_CH, CHUNK), jnp.int32),
        pltpu.VMEM((2, E_PER_SG), jnp.float32),
        pltpu.VMEM((NBUF, CHUNK, FT), jnp.float32),
        pltpu.SemaphoreType.DMA((NBUF,)),
        pltpu.SemaphoreType.DMA((NBUF,)),
        pltpu.SemaphoreType.DMA((2,)),
        pltpu.SemaphoreType.DMA((2,)),
        pltpu.SemaphoreType.DMA((2,)),
    ],
)


def _mm_body(p0_ref, p1_ref, w_ref, o_ref):
  p = p0_ref[...] + p1_ref[...]
  o_ref[...] = jnp.maximum(
      jnp.dot(p, w_ref[...], preferred_element_type=jnp.float32), 0.0)


def _matmul_relu(p0, p1, W):
  blk = 1000
  return pl.pallas_call(
      _mm_body,
      grid=(N_NODES // blk,),
      in_specs=[
          pl.BlockSpec((blk, FT), lambda i: (i, 0)),
          pl.BlockSpec((blk, FT), lambda i: (i, 0)),
          pl.BlockSpec((FT, FT), lambda i: (0, 0)),
      ],
      out_specs=pl.BlockSpec((blk, FT), lambda i: (i, 0)),
      out_shape=jax.ShapeDtypeStruct((N_NODES, FT), jnp.float32),
  )(p0, p1, W)


@jax.jit
def kernel(seq, edge_index, edge_weight, W):
  n_edges = edge_index.shape[1]
  src = edge_index[0].astype(jnp.int32)
  dst = edge_index[1].astype(jnp.int32)
  w = edge_weight.astype(jnp.float32)

  pad = EPAD - n_edges
  src = jnp.pad(src, (0, pad)).reshape(NW, NSG, SG_CH, CHUNK)
  dst = jnp.pad(dst, (0, pad)).reshape(NW, NSG, SG_CH, CHUNK)
  w = jnp.pad(w, (0, pad)).reshape(NW, NSG, E_PER_SG)

  zeros = jnp.zeros((NPAD, FT), jnp.float32)
  partial = _spmm(seq, src, dst, w, zeros)
  return _matmul_relu(partial[0], partial[1], W)
